# baseline (speedup 1.0000x reference)
import jax
import jax.numpy as jnp
from jax import lax
from jax.experimental import pallas as pl
from jax.experimental.pallas import tpu as pltpu
from jax.experimental.pallas import tpu_sc as plsc

_NCODES = 1024
_LDIM = 32
_LANES = 16
_SUB = 8
_NWORK = 32


def _argmin_body(z_ref, cb_ref, idx4_ref, idxf_ref, zc_ref, z_s):
    d, h, w = z_ref.shape[1:]
    t = h * w
    for j in range(h):
        z_s[:, pl.ds(j * w, w)] = z_ref[0, :, j, :]
    z = z_s[...]
    cb = cb_ref[...]
    e2 = jnp.sum(cb * cb, axis=1)
    scores = lax.dot_general(
        cb, z, (((1,), (0,)), ((), ())),
        preferred_element_type=jnp.float32,
    )
    dist = e2[:, None] - 2.0 * scores
    idx = jnp.argmin(dist, axis=0).astype(jnp.int32)
    for j in range(h):
        idx4_ref[0, j, :] = idx[j * w:(j + 1) * w]
    idxf_ref[...] = idx
    for r in range(d // _SUB):
        for c in range(t // 128):
            zc_ref[r, c] = z[r * _SUB:(r + 1) * _SUB, c * 128:(c + 1) * 128]


def _make_gather_body(imgs):
    wpi = _NWORK // imgs
    tw = _NCODES // wpi
    cw = tw // 128

    def body(cbt_hbm, idx_hbm, zc_hbm, ste_hbm, delta_hbm,
             cbt_v, idx_v, ek_v, zvc, dvw):
        wid = lax.axis_index("s") * 2 + lax.axis_index("c")
        img = wid // wpi
        part = wid % wpi
        pltpu.sync_copy(cbt_hbm, cbt_v)
        pltpu.sync_copy(idx_hbm.at[pl.ds(wid * tw, tw)], idx_v)
        rows = [jnp.full((_LANES,), d, jnp.int32) for d in range(_LDIM)]

        def group(g, _):
            base = g * _LANES
            idx16 = idx_v[pl.ds(base, _LANES)]
            for d in range(_LDIM):
                ek_v[d, pl.ds(base, _LANES)] = plsc.load_gather(
                    cbt_v, [rows[d], idx16])
            return ()

        lax.fori_loop(0, tw // _LANES, group, (), unroll=2)
        pltpu.sync_copy(ek_v, ste_hbm.at[img, :, pl.ds(part * tw, tw)])

        for dg in range(_LDIM // _SUB):
            pltpu.sync_copy(zc_hbm.at[dg, pl.ds(wid * cw, cw)], zvc)

            def dgrp(cq, _):
                tok = cq * 128
                for li in range(_SUB):
                    lo = li * _LANES
                    for s in range(_SUB):
                        zvec = zvc[cq, s, pl.ds(lo, _LANES)]
                        evec = ek_v[dg * _SUB + s, pl.ds(tok + lo, _LANES)]
                        dvw[s, pl.ds(tok + lo, _LANES)] = zvec - evec
                return ()

            lax.fori_loop(0, cw, dgrp, (), unroll=2)
            pltpu.sync_copy(
                dvw, delta_hbm.at[img, pl.ds(dg * _SUB, _SUB),
                                  pl.ds(part * tw, tw)])

    return body, tw, cw


def kernel(z_e, codebook):
    b, d, h, w = z_e.shape
    t = h * w
    half = b // 2
    rt, ct = d // _SUB, (half * t) // 128

    def stage_a(offset):
        return pl.pallas_call(
            _argmin_body,
            grid=(half,),
            in_specs=[
                pl.BlockSpec((1, d, h, w), lambda i: (i + offset, 0, 0, 0)),
                pl.BlockSpec((_NCODES, _LDIM), lambda i: (0, 0)),
            ],
            out_specs=[
                pl.BlockSpec((1, h, w), lambda i: (i, 0, 0)),
                pl.BlockSpec((t,), lambda i: (i,)),
                pl.BlockSpec((rt, _SUB, _SUB, 128), lambda i: (0, i, 0, 0)),
            ],
            out_shape=[
                jax.ShapeDtypeStruct((half, h, w), jnp.int32),
                jax.ShapeDtypeStruct((half * t,), jnp.int32),
                jax.ShapeDtypeStruct((rt, ct, _SUB, 128), jnp.float32),
            ],
            scratch_shapes=[pltpu.VMEM((d, t), jnp.float32)],
        )(z_e, codebook)

    idx4_a, idxf_a, zc_a = stage_a(0)
    idx4_b, idxf_b, zc_b = stage_a(half)

    gbody, tw, cw = _make_gather_body(half)
    cbt = codebook.T
    sc_gather = pl.kernel(
        gbody,
        mesh=plsc.VectorSubcoreMesh(core_axis_name="c", subcore_axis_name="s"),
        compiler_params=pltpu.CompilerParams(
            use_tc_tiling_on_sc=False, needs_layout_passes=False
        ),
        out_type=[
            jax.ShapeDtypeStruct((half, d, t), jnp.float32),
            jax.ShapeDtypeStruct((half, d, t), jnp.float32),
        ],
        scratch_types=[
            pltpu.VMEM((d, _NCODES), jnp.float32),
            pltpu.VMEM((tw,), jnp.int32),
            pltpu.VMEM((d, tw), jnp.float32),
            pltpu.VMEM((cw, _SUB, 128), jnp.float32),
            pltpu.VMEM((_SUB, tw), jnp.float32),
        ],
    )
    ste_a, delta_a = sc_gather(cbt, idxf_a, zc_a)
    ste_b, delta_b = sc_gather(cbt, idxf_b, zc_b)

    ste = jnp.concatenate([ste_a, ste_b], 0).reshape(b, d, h, w)
    delta = jnp.concatenate([delta_a, delta_b], 0).reshape(b, d, h, w)
    idx4 = jnp.concatenate([idx4_a, idx4_b], 0)
    return (ste, idx4, delta)


# speedup vs baseline: 1.0748x; 1.0748x over previous
import jax
import jax.numpy as jnp
from jax import lax
from jax.experimental import pallas as pl
from jax.experimental.pallas import tpu as pltpu
from jax.experimental.pallas import tpu_sc as plsc

_NCODES = 1024
_LDIM = 32
_LANES = 16
_SUB = 8


def _argmin_body(z_ref, cb_ref, idx4_ref, idxf_ref, zc_ref, z_s):
    d, h, w = z_ref.shape[1:]
    t = h * w
    for j in range(h):
        z_s[:, pl.ds(j * w, w)] = z_ref[0, :, j, :]
    z = z_s[...]
    cb = cb_ref[...]
    e2 = jnp.sum(cb * cb, axis=1)
    scores = lax.dot_general(
        cb, z, (((1,), (0,)), ((), ())),
        preferred_element_type=jnp.float32,
    )
    dist = e2[:, None] - 2.0 * scores
    idx = jnp.argmin(dist, axis=0).astype(jnp.int32)
    for j in range(h):
        idx4_ref[0, j, :] = idx[j * w:(j + 1) * w]
    idxf_ref[...] = idx
    for r in range(d // _SUB):
        for c in range(t // 128):
            zc_ref[r, c] = z[r * _SUB:(r + 1) * _SUB, c * 128:(c + 1) * 128]


def _gather_body(cbt_hbm, idx_hbm, zc_hbm, ste_hbm, delta_hbm,
                 cbt_v, idx_v, ek_v, zv8, dv8):
    wid = lax.axis_index("s") * 2 + lax.axis_index("c")
    t = _NCODES
    pltpu.sync_copy(cbt_hbm, cbt_v)
    pltpu.sync_copy(idx_hbm.at[pl.ds(wid * t, t)], idx_v)

    def group(g, _):
        base = g * _LANES
        idx16 = idx_v[pl.ds(base, _LANES)]
        for d in range(_LDIM):
            row = jnp.full((_LANES,), d, jnp.int32)
            ek_v[d, pl.ds(base, _LANES)] = plsc.load_gather(cbt_v, [row, idx16])
        return ()

    lax.fori_loop(0, t // _LANES, group, (), unroll=2)
    pltpu.sync_copy(ek_v, ste_hbm.at[wid])

    for dg in range(_LDIM // _SUB):
        pltpu.sync_copy(zc_hbm.at[dg, pl.ds(wid * _SUB, _SUB)], zv8)

        def dgrp(g, _):
            cq = g // _SUB
            lo = (g % _SUB) * _LANES
            for s in range(_SUB):
                zvec = zv8[cq, s, pl.ds(lo, _LANES)]
                evec = ek_v[dg * _SUB + s, pl.ds(g * _LANES, _LANES)]
                dv8[s, pl.ds(g * _LANES, _LANES)] = zvec - evec
            return ()

        lax.fori_loop(0, t // _LANES, dgrp, (), unroll=2)
        pltpu.sync_copy(dv8, delta_hbm.at[wid, pl.ds(dg * _SUB, _SUB)])


def kernel(z_e, codebook):
    b, d, h, w = z_e.shape
    t = h * w
    rt, ct = d // _SUB, (b * t) // 128

    idx4, idxf, zc = pl.pallas_call(
        _argmin_body,
        grid=(b,),
        in_specs=[
            pl.BlockSpec((1, d, h, w), lambda i: (i, 0, 0, 0)),
            pl.BlockSpec((_NCODES, _LDIM), lambda i: (0, 0)),
        ],
        out_specs=[
            pl.BlockSpec((1, h, w), lambda i: (i, 0, 0)),
            pl.BlockSpec((t,), lambda i: (i,)),
            pl.BlockSpec((rt, _SUB, _SUB, 128), lambda i: (0, i, 0, 0)),
        ],
        out_shape=[
            jax.ShapeDtypeStruct((b, h, w), jnp.int32),
            jax.ShapeDtypeStruct((b * t,), jnp.int32),
            jax.ShapeDtypeStruct((rt, ct, _SUB, 128), jnp.float32),
        ],
        scratch_shapes=[pltpu.VMEM((d, t), jnp.float32)],
    )(z_e, codebook)

    cbt = codebook.T
    sc_gather = pl.kernel(
        _gather_body,
        mesh=plsc.VectorSubcoreMesh(core_axis_name="c", subcore_axis_name="s"),
        compiler_params=pltpu.CompilerParams(
            use_tc_tiling_on_sc=False, needs_layout_passes=False
        ),
        out_type=[
            jax.ShapeDtypeStruct((b, d, t), jnp.float32),
            jax.ShapeDtypeStruct((b, d, t), jnp.float32),
        ],
        scratch_types=[
            pltpu.VMEM((d, _NCODES), jnp.float32),
            pltpu.VMEM((t,), jnp.int32),
            pltpu.VMEM((d, t), jnp.float32),
            pltpu.VMEM((_SUB, _SUB, 128), jnp.float32),
            pltpu.VMEM((_SUB, t), jnp.float32),
        ],
    )
    ste_l, delta_l = sc_gather(cbt, idxf, zc)

    return (
        ste_l.reshape(b, d, h, w),
        idx4,
        delta_l.reshape(b, d, h, w),
    )


# speedup vs baseline: 1.0955x; 1.0193x over previous
import jax
import jax.numpy as jnp
from jax import lax
from jax.experimental import pallas as pl
from jax.experimental.pallas import tpu as pltpu
from jax.experimental.pallas import tpu_sc as plsc

_NCODES = 1024
_LDIM = 32
_LANES = 16
_SUB = 8


def _argmin_body(z_ref, cb_ref, idx4_ref, idxf_ref, zc_ref, z_s):
    d, h, w = z_ref.shape[1:]
    t = h * w
    for j in range(h):
        z_s[:, pl.ds(j * w, w)] = z_ref[0, :, j, :]
    z = z_s[...]
    cb = cb_ref[...]
    e2 = jnp.sum(cb * cb, axis=1)
    scores = lax.dot_general(
        cb, z, (((1,), (0,)), ((), ())),
        preferred_element_type=jnp.float32,
    )
    dist = e2[:, None] - 2.0 * scores
    idx = jnp.argmin(dist, axis=0).astype(jnp.int32)
    for j in range(h):
        idx4_ref[0, j, :] = idx[j * w:(j + 1) * w]
    idxf_ref[...] = idx
    for r in range(d // _SUB):
        for c in range(t // 128):
            zc_ref[r, c] = z[r * _SUB:(r + 1) * _SUB, c * 128:(c + 1) * 128]


def _gather_body(cbt_hbm, idx_hbm, zc_hbm, ste_hbm, delta_hbm,
                 cbt_v, idx_v, ek_v, z_v, dv_v):
    wid = lax.axis_index("s") * 2 + lax.axis_index("c")
    t = _NCODES
    pltpu.sync_copy(cbt_hbm, cbt_v)
    pltpu.sync_copy(idx_hbm.at[pl.ds(wid * t, t)], idx_v)
    pltpu.sync_copy(zc_hbm.at[:, pl.ds(wid * _SUB, _SUB)], z_v)
    rows = [jnp.full((_LANES,), d, jnp.int32) for d in range(_LDIM)]

    th = t // 2

    for p in range(2):
        def group(g, _):
            base = g * _LANES
            cq = g // _SUB
            lo = (g % _SUB) * _LANES
            idx16 = idx_v[pl.ds(base, _LANES)]
            for d in range(_LDIM):
                val = plsc.load_gather(cbt_v, [rows[d], idx16])
                ek_v[d, pl.ds(base, _LANES)] = val
                zvec = z_v[d // _SUB, cq, d % _SUB, pl.ds(lo, _LANES)]
                dv_v[d, pl.ds(base - p * th, _LANES)] = zvec - val
            return ()

        lax.fori_loop(p * (th // _LANES), (p + 1) * (th // _LANES),
                      group, (), unroll=2)
        pltpu.sync_copy(dv_v, delta_hbm.at[wid, :, pl.ds(p * th, th)])

    pltpu.sync_copy(ek_v, ste_hbm.at[wid])


def kernel(z_e, codebook):
    b, d, h, w = z_e.shape
    t = h * w
    rt, ct = d // _SUB, (b * t) // 128

    idx4, idxf, zc = pl.pallas_call(
        _argmin_body,
        grid=(b,),
        in_specs=[
            pl.BlockSpec((1, d, h, w), lambda i: (i, 0, 0, 0)),
            pl.BlockSpec((_NCODES, _LDIM), lambda i: (0, 0)),
        ],
        out_specs=[
            pl.BlockSpec((1, h, w), lambda i: (i, 0, 0)),
            pl.BlockSpec((t,), lambda i: (i,)),
            pl.BlockSpec((rt, _SUB, _SUB, 128), lambda i: (0, i, 0, 0)),
        ],
        out_shape=[
            jax.ShapeDtypeStruct((b, h, w), jnp.int32),
            jax.ShapeDtypeStruct((b * t,), jnp.int32),
            jax.ShapeDtypeStruct((rt, ct, _SUB, 128), jnp.float32),
        ],
        scratch_shapes=[pltpu.VMEM((d, t), jnp.float32)],
    )(z_e, codebook)

    cbt = codebook.T
    sc_gather = pl.kernel(
        _gather_body,
        mesh=plsc.VectorSubcoreMesh(core_axis_name="c", subcore_axis_name="s"),
        compiler_params=pltpu.CompilerParams(
            use_tc_tiling_on_sc=False, needs_layout_passes=False
        ),
        out_type=[
            jax.ShapeDtypeStruct((b, d, t), jnp.float32),
            jax.ShapeDtypeStruct((b, d, t), jnp.float32),
        ],
        scratch_types=[
            pltpu.VMEM((d, _NCODES), jnp.float32),
            pltpu.VMEM((t,), jnp.int32),
            pltpu.VMEM((d, t), jnp.float32),
            pltpu.VMEM((rt, _SUB, _SUB, 128), jnp.float32),
            pltpu.VMEM((d, t // 2), jnp.float32),
        ],
    )
    ste_l, delta_l = sc_gather(cbt, idxf, zc)

    return (
        ste_l.reshape(b, d, h, w),
        idx4,
        delta_l.reshape(b, d, h, w),
    )


# speedup vs baseline: 1.1041x; 1.0078x over previous
import jax
import jax.numpy as jnp
from jax import lax
from jax.experimental import pallas as pl
from jax.experimental.pallas import tpu as pltpu
from jax.experimental.pallas import tpu_sc as plsc

_NCODES = 1024
_LDIM = 32
_LANES = 16
_SUB = 8


def _argmin_body(z_ref, cb_ref, idx4_ref, idxf_ref, zc_ref, z_s):
    d, h, w = z_ref.shape[1:]
    t = h * w
    for j in range(h):
        z_s[:, pl.ds(j * w, w)] = z_ref[0, :, j, :]
    z = z_s[...]
    cb = cb_ref[...]
    e2 = jnp.sum(cb * cb, axis=1)
    scores = lax.dot_general(
        cb, z, (((1,), (0,)), ((), ())),
        preferred_element_type=jnp.float32,
    )
    dist = e2[:, None] - 2.0 * scores
    idx = jnp.argmin(dist, axis=0).astype(jnp.int32)
    for j in range(h):
        idx4_ref[0, j, :] = idx[j * w:(j + 1) * w]
    idxf_ref[...] = idx
    for r in range(d // _SUB):
        for c in range(t // 128):
            zc_ref[r, c] = z[r * _SUB:(r + 1) * _SUB, c * 128:(c + 1) * 128]


def _gather_body(cbt_hbm, idx_hbm, zc_hbm, ste_hbm, delta_hbm,
                 cbt_v, idx_v, ek_v, z_v, dv_v, sem):
    wid = lax.axis_index("s") * 2 + lax.axis_index("c")
    t = _NCODES
    c1 = pltpu.async_copy(cbt_hbm, cbt_v, sem)
    c2 = pltpu.async_copy(idx_hbm.at[pl.ds(wid * t, t)], idx_v, sem)
    c3 = pltpu.async_copy(zc_hbm.at[:, pl.ds(wid * _SUB, _SUB)], z_v, sem)
    c1.wait()
    c2.wait()
    c3.wait()
    rows = [jnp.full((_LANES,), d, jnp.int32) for d in range(_LDIM)]

    th = t // 2

    for p in range(2):
        def group(g, _):
            base = g * _LANES
            cq = g // _SUB
            lo = (g % _SUB) * _LANES
            idx16 = idx_v[pl.ds(base, _LANES)]
            for d in range(_LDIM):
                val = plsc.load_gather(cbt_v, [rows[d], idx16])
                ek_v[d, pl.ds(base, _LANES)] = val
                zvec = z_v[d // _SUB, cq, d % _SUB, pl.ds(lo, _LANES)]
                dv_v[d, pl.ds(base - p * th, _LANES)] = zvec - val
            return ()

        lax.fori_loop(p * (th // _LANES), (p + 1) * (th // _LANES),
                      group, (), unroll=2)
        pltpu.sync_copy(dv_v, delta_hbm.at[wid, :, pl.ds(p * th, th)])

    pltpu.sync_copy(ek_v, ste_hbm.at[wid])


def kernel(z_e, codebook):
    b, d, h, w = z_e.shape
    t = h * w
    rt, ct = d // _SUB, (b * t) // 128

    idx4, idxf, zc = pl.pallas_call(
        _argmin_body,
        grid=(b,),
        in_specs=[
            pl.BlockSpec((1, d, h, w), lambda i: (i, 0, 0, 0)),
            pl.BlockSpec((_NCODES, _LDIM), lambda i: (0, 0)),
        ],
        out_specs=[
            pl.BlockSpec((1, h, w), lambda i: (i, 0, 0)),
            pl.BlockSpec((t,), lambda i: (i,)),
            pl.BlockSpec((rt, _SUB, _SUB, 128), lambda i: (0, i, 0, 0)),
        ],
        out_shape=[
            jax.ShapeDtypeStruct((b, h, w), jnp.int32),
            jax.ShapeDtypeStruct((b * t,), jnp.int32),
            jax.ShapeDtypeStruct((rt, ct, _SUB, 128), jnp.float32),
        ],
        scratch_shapes=[pltpu.VMEM((d, t), jnp.float32)],
    )(z_e, codebook)

    cbt = codebook.T
    sc_gather = pl.kernel(
        _gather_body,
        mesh=plsc.VectorSubcoreMesh(core_axis_name="c", subcore_axis_name="s"),
        compiler_params=pltpu.CompilerParams(
            use_tc_tiling_on_sc=False, needs_layout_passes=False
        ),
        out_type=[
            jax.ShapeDtypeStruct((b, d, t), jnp.float32),
            jax.ShapeDtypeStruct((b, d, t), jnp.float32),
        ],
        scratch_types=[
            pltpu.VMEM((d, _NCODES), jnp.float32),
            pltpu.VMEM((t,), jnp.int32),
            pltpu.VMEM((d, t), jnp.float32),
            pltpu.VMEM((rt, _SUB, _SUB, 128), jnp.float32),
            pltpu.VMEM((d, t // 2), jnp.float32),
            pltpu.SemaphoreType.DMA,
        ],
    )
    ste_l, delta_l = sc_gather(cbt, idxf, zc)

    return (
        ste_l.reshape(b, d, h, w),
        idx4,
        delta_l.reshape(b, d, h, w),
    )


# speedup vs baseline: 1.1117x; 1.0068x over previous
import jax
import jax.numpy as jnp
from jax import lax
from jax.experimental import pallas as pl
from jax.experimental.pallas import tpu as pltpu
from jax.experimental.pallas import tpu_sc as plsc

_NCODES = 1024
_LDIM = 32
_LANES = 16
_SUB = 8


def _argmin_body(z_ref, cb_ref, idx4_ref, idxf_ref, zc_ref, z_s):
    d, h, w = z_ref.shape[1:]
    t = h * w
    for j in range(h):
        z_s[:, pl.ds(j * w, w)] = z_ref[0, :, j, :]
    z = z_s[...]
    cb = cb_ref[...]
    e2 = jnp.sum(cb * cb, axis=1)
    scores = lax.dot_general(
        cb, z, (((1,), (0,)), ((), ())),
        preferred_element_type=jnp.float32,
    )
    dist = e2[:, None] - 2.0 * scores
    idx = jnp.argmin(dist, axis=0).astype(jnp.int32)
    for j in range(h):
        idx4_ref[0, j, :] = idx[j * w:(j + 1) * w]
    idxf_ref[...] = idx
    for r in range(d // _SUB):
        for c in range(t // 128):
            zc_ref[r, c] = z[r * _SUB:(r + 1) * _SUB, c * 128:(c + 1) * 128]


def _gather_body(cbt_hbm, idx_hbm, zc_hbm, ste_hbm, delta_hbm,
                 cbt_v, idx_v, ek_v, z_v, dv_v, sem):
    wid = lax.axis_index("s") * 2 + lax.axis_index("c")
    t = _NCODES
    c1 = pltpu.async_copy(cbt_hbm, cbt_v, sem)
    c2 = pltpu.async_copy(idx_hbm.at[pl.ds(wid * t, t)], idx_v, sem)
    c3 = pltpu.async_copy(zc_hbm.at[:, pl.ds(wid * _SUB, _SUB)], z_v, sem)
    c1.wait()
    c2.wait()
    c3.wait()
    rows = [jnp.full((_LANES,), d, jnp.int32) for d in range(_LDIM)]

    th = t // 2

    def make_pass(p):
        def group(g, _):
            base = g * _LANES
            cq = g // _SUB
            lo = (g % _SUB) * _LANES
            idx16 = idx_v[pl.ds(base, _LANES)]
            for d in range(_LDIM):
                val = plsc.load_gather(cbt_v, [rows[d], idx16])
                ek_v[d, pl.ds(base, _LANES)] = val
                zvec = z_v[d // _SUB, cq, d % _SUB, pl.ds(lo, _LANES)]
                dv_v[d, pl.ds(base - p * th, _LANES)] = zvec - val
            return ()

        lax.fori_loop(p * (th // _LANES), (p + 1) * (th // _LANES),
                      group, (), unroll=2)

    make_pass(0)
    d1 = pltpu.async_copy(dv_v, delta_hbm.at[wid, :, pl.ds(0, th)], sem)
    e1 = pltpu.async_copy(ek_v.at[:, pl.ds(0, th)],
                          ste_hbm.at[wid, :, pl.ds(0, th)], sem)
    d1.wait()
    make_pass(1)
    d2 = pltpu.async_copy(dv_v, delta_hbm.at[wid, :, pl.ds(th, th)], sem)
    e2 = pltpu.async_copy(ek_v.at[:, pl.ds(th, th)],
                          ste_hbm.at[wid, :, pl.ds(th, th)], sem)
    e1.wait()
    d2.wait()
    e2.wait()


def kernel(z_e, codebook):
    b, d, h, w = z_e.shape
    t = h * w
    rt, ct = d // _SUB, (b * t) // 128

    idx4, idxf, zc = pl.pallas_call(
        _argmin_body,
        grid=(b,),
        in_specs=[
            pl.BlockSpec((1, d, h, w), lambda i: (i, 0, 0, 0)),
            pl.BlockSpec((_NCODES, _LDIM), lambda i: (0, 0)),
        ],
        out_specs=[
            pl.BlockSpec((1, h, w), lambda i: (i, 0, 0)),
            pl.BlockSpec((t,), lambda i: (i,)),
            pl.BlockSpec((rt, _SUB, _SUB, 128), lambda i: (0, i, 0, 0)),
        ],
        out_shape=[
            jax.ShapeDtypeStruct((b, h, w), jnp.int32),
            jax.ShapeDtypeStruct((b * t,), jnp.int32),
            jax.ShapeDtypeStruct((rt, ct, _SUB, 128), jnp.float32),
        ],
        scratch_shapes=[pltpu.VMEM((d, t), jnp.float32)],
    )(z_e, codebook)

    cbt = codebook.T
    sc_gather = pl.kernel(
        _gather_body,
        mesh=plsc.VectorSubcoreMesh(core_axis_name="c", subcore_axis_name="s"),
        compiler_params=pltpu.CompilerParams(
            use_tc_tiling_on_sc=False, needs_layout_passes=False
        ),
        out_type=[
            jax.ShapeDtypeStruct((b, d, t), jnp.float32),
            jax.ShapeDtypeStruct((b, d, t), jnp.float32),
        ],
        scratch_types=[
            pltpu.VMEM((d, _NCODES), jnp.float32),
            pltpu.VMEM((t,), jnp.int32),
            pltpu.VMEM((d, t), jnp.float32),
            pltpu.VMEM((rt, _SUB, _SUB, 128), jnp.float32),
            pltpu.VMEM((d, t // 2), jnp.float32),
            pltpu.SemaphoreType.DMA,
        ],
    )
    ste_l, delta_l = sc_gather(cbt, idxf, zc)

    return (
        ste_l.reshape(b, d, h, w),
        idx4,
        delta_l.reshape(b, d, h, w),
    )
